# Initial kernel scaffold; baseline (speedup 1.0000x reference)
import jax
import jax.numpy as jnp
from jax.experimental import pallas as pl


def kernel(x, score):
    raise NotImplementedError("write your pallas kernel here")


# speedup vs baseline: 102.3133x; 102.3133x over previous
import jax
import jax.numpy as jnp
from jax.experimental import pallas as pl

_BLK = 8
_KEEP = 4
_TM = 128


def _roll(s, k):
    n = s.shape[-1]
    k = k % n
    if k == 0:
        return s
    return jnp.concatenate([s[:, n - k:], s[:, : n - k]], axis=1)


def _mask_kernel(x_ref, s_ref, o_ref):
    s = s_ref[...]
    x = x_ref[...]
    lane = jax.lax.broadcasted_iota(jnp.int32, s.shape, 1) % _BLK
    rank = jnp.zeros(s.shape, jnp.float32)
    for k in range(1, _BLK):
        r1 = _roll(s, k)
        r2 = _roll(s, k - _BLK)
        pred = lane >= k
        cmp = (pred & (r1 <= s)) | ((~pred) & (r2 < s))
        rank = rank + jnp.where(cmp, 1.0, 0.0)
    o_ref[...] = jnp.where(rank >= float(_KEEP), x, 0.0)


def kernel(x, score):
    m, n = x.shape
    grid = (m // _TM,)
    spec = pl.BlockSpec((_TM, n), lambda i: (i, 0))
    return pl.pallas_call(
        _mask_kernel,
        grid=grid,
        in_specs=[spec, spec],
        out_specs=spec,
        out_shape=jax.ShapeDtypeStruct((m, n), x.dtype),
    )(x, score)


# speedup vs baseline: 132.1538x; 1.2917x over previous
import jax
import jax.numpy as jnp
from jax.experimental import pallas as pl

_BLK = 8
_KEEP = 4
_TM = 4096
_TN = 128


def _wgroll(a, k):
    n = a.shape[-1]
    idx = (jnp.arange(n) // _BLK) * _BLK + (jnp.arange(n) - k) % _BLK
    idx = jnp.broadcast_to(idx[None, :], a.shape)
    return jnp.take_along_axis(a, idx, axis=-1)


def _mask_kernel(x_ref, s_ref, o_ref):
    s = s_ref[...]
    x = x_ref[...]
    lane = jax.lax.broadcasted_iota(jnp.int32, s.shape, 1) % _BLK
    rank = jnp.zeros(s.shape, jnp.float32)
    for k in range(1, 5):
        t = _wgroll(s, k)
        pred = lane >= k
        c = (t < s) | (pred & (t == s))
        cf = jnp.where(c, 1.0, 0.0)
        rank = rank + cf
        if k < 4:
            rank = rank - _wgroll(cf, -k)
    o_ref[...] = jnp.where(rank >= 1.0, x, 0.0)


def kernel(x, score):
    m, n = x.shape
    xr = x.reshape(-1, _TN)
    sr = score.reshape(-1, _TN)
    m2 = xr.shape[0]
    grid = (m2 // _TM,)
    spec = pl.BlockSpec((_TM, _TN), lambda i: (i, 0))
    out = pl.pallas_call(
        _mask_kernel,
        grid=grid,
        in_specs=[spec, spec],
        out_specs=spec,
        out_shape=jax.ShapeDtypeStruct((m2, _TN), x.dtype),
    )(xr, sr)
    return out.reshape(m, n)


# speedup vs baseline: 355.9743x; 2.6936x over previous
import jax
import jax.numpy as jnp
from jax.experimental import pallas as pl

_BLK = 8
_KEEP = 4
_TM = 4096
_TN = 128


def _wgroll(a, k):
    n = a.shape[-1]
    idx = (jnp.arange(n) // _BLK) * _BLK + (jnp.arange(n) - k) % _BLK
    idx = jnp.broadcast_to(idx[None, :], a.shape)
    return jnp.take_along_axis(a, idx, axis=-1)


def _mask_kernel(x_ref, s_ref, o_ref):
    s = s_ref[...]
    x = x_ref[...]
    lane = jax.lax.broadcasted_iota(jnp.int32, s.shape, 1) % _BLK
    rank = jnp.zeros(s.shape, jnp.float32)
    for k in range(1, 5):
        t = _wgroll(s, k)
        pred = lane >= k
        c = (t < s) | (pred & (t == s))
        cf = jnp.where(c, 1.0, 0.0)
        rank = rank + cf
        if k < 4:
            rank = rank - _wgroll(cf, -k)
    o_ref[...] = jnp.where(rank >= 1.0, x, 0.0)


def kernel(x, score):
    m, n = x.shape
    grid = (m // _TM, n // _TN)
    spec = pl.BlockSpec((_TM, _TN), lambda i, j: (i, j))
    return pl.pallas_call(
        _mask_kernel,
        grid=grid,
        in_specs=[spec, spec],
        out_specs=spec,
        out_shape=jax.ShapeDtypeStruct((m, n), x.dtype),
    )(x, score)


# speedup vs baseline: 423.7166x; 1.1903x over previous
import jax
import jax.numpy as jnp
from jax.experimental import pallas as pl

_BLK = 8
_KEEP = 4
_TM = 4096
_TN = 128


def _wgroll(a, k):
    n = a.shape[-1]
    idx = (jnp.arange(n) // _BLK) * _BLK + (jnp.arange(n) - k) % _BLK
    idx = jnp.broadcast_to(idx[None, :], a.shape)
    return jnp.take_along_axis(a, idx, axis=-1)


def _mask_kernel(x_ref, s_ref, o_ref):
    s = s_ref[...]
    x = x_ref[...]
    b = jax.lax.bitcast_convert_type(jnp.where(s == 0.0, 0.0, s), jnp.int32)
    key = b ^ jax.lax.shift_right_logical(jax.lax.shift_right_arithmetic(b, 31), 1)
    key1 = key + 1
    lane = jax.lax.broadcasted_iota(jnp.int32, s.shape, 1) % _BLK
    rank = jnp.zeros(s.shape, jnp.float32)
    for k in range(1, 5):
        t = _wgroll(key, k)
        c = t < jnp.where(lane >= k, key1, key)
        cf = jnp.where(c, 1.0, 0.0)
        rank = rank + cf
        if k < 4:
            rank = rank - _wgroll(cf, -k)
    o_ref[...] = jnp.where(rank >= 1.0, x, 0.0)


def kernel(x, score):
    m, n = x.shape
    grid = (m // _TM, n // _TN)
    spec = pl.BlockSpec((_TM, _TN), lambda i, j: (i, j))
    return pl.pallas_call(
        _mask_kernel,
        grid=grid,
        in_specs=[spec, spec],
        out_specs=spec,
        out_shape=jax.ShapeDtypeStruct((m, n), x.dtype),
    )(x, score)
